# baseline (speedup 1.0000x reference)
import functools

import jax
import jax.numpy as jnp
from jax import lax
from jax.experimental import pallas as pl
from jax.experimental.pallas import tpu as pltpu
from jax.experimental.pallas import tpu_sc as plsc

_EPS = 1e-5
_B_BLK = 256
_NPAD = 128
_ROWS_PER_W = 8


def _sc_gather(indices_pad, W):
    n_rows, dim = W.shape
    mesh = plsc.VectorSubcoreMesh(core_axis_name="c", subcore_axis_name="s")
    info = plsc.get_sparse_core_info()
    num_cores = info.num_cores

    @functools.partial(
        pl.kernel,
        mesh=mesh,
        out_type=jax.ShapeDtypeStruct((_NPAD, dim), jnp.float32),
        scratch_types=[
            pltpu.VMEM((_ROWS_PER_W,), jnp.int32),
            pltpu.VMEM((_ROWS_PER_W, dim), jnp.float32),
            pltpu.SemaphoreType.DMA,
        ],
    )
    def gather(idx_hbm, w_hbm, out_hbm, idx_v, rows_v, sem):
        wid = lax.axis_index("s") * num_cores + lax.axis_index("c")

        @pl.when(wid < _NPAD // _ROWS_PER_W)
        def _():
            base = wid * _ROWS_PER_W
            pltpu.sync_copy(idx_hbm.at[pl.ds(base, _ROWS_PER_W)], idx_v)
            pltpu.async_copy(w_hbm.at[idx_v], rows_v, sem).wait()
            pltpu.sync_copy(rows_v, out_hbm.at[pl.ds(base, _ROWS_PER_W)])

    return gather(indices_pad, W)


def _dense_body(v_ref, rows_ref, par_ref, g_ref, b_ref, out_ref):
    rows = rows_ref[...]
    par = par_ref[...]
    dim = rows.shape[-1] // 2
    sel = jnp.where(par > 0, rows[:, dim:], rows[:, :dim])
    mean = jnp.mean(sel, axis=-1, keepdims=True)
    cent = sel - mean
    var = jnp.mean(cent * cent, axis=-1)
    c = cent * g_ref[...]
    v = v_ref[...]
    s = v * lax.rsqrt(v * v * var[None, :] + _EPS)
    out_ref[...] = s[:, :, None] * c[None] + b_ref[...][None]


def kernel(values, indices, W, ln_gamma, ln_beta):
    batch, n_fields = values.shape
    n_rows, dim = W.shape
    idx = jnp.zeros((_NPAD,), jnp.int32).at[:n_fields].set(
        indices.astype(jnp.int32))
    w2 = W.reshape(n_rows // 2, 2 * dim)
    rows = _sc_gather(idx >> 1, w2)[:n_fields]
    par = (idx[:n_fields] & 1).reshape(n_fields, 1)

    grid = batch // _B_BLK
    out = pl.pallas_call(
        _dense_body,
        grid=(grid,),
        in_specs=[
            pl.BlockSpec((_B_BLK, n_fields), lambda i: (i, 0)),
            pl.BlockSpec((n_fields, 2 * dim), lambda i: (0, 0)),
            pl.BlockSpec((n_fields, 1), lambda i: (0, 0)),
            pl.BlockSpec((1, dim), lambda i: (0, 0)),
            pl.BlockSpec((1, dim), lambda i: (0, 0)),
        ],
        out_specs=pl.BlockSpec((_B_BLK, n_fields, dim), lambda i: (i, 0, 0)),
        out_shape=jax.ShapeDtypeStruct((batch, n_fields, dim), jnp.float32),
    )(values, rows, par, ln_gamma.reshape(1, dim), ln_beta.reshape(1, dim))
    return out


# speedup vs baseline: 1.5882x; 1.5882x over previous
import jax
import jax.numpy as jnp
from jax import lax
from jax.experimental import pallas as pl
from jax.experimental.pallas import tpu as pltpu

_EPS = 1e-5
_B_BLK = 256


def _body(idx_ref, v_ref, w_hbm, g_ref, b_ref, out_ref, sel_ref, sem):
    n_fields = sel_ref.shape[0]

    @pl.when(pl.program_id(0) == 0)
    def _gather():
        for i in range(n_fields):
            pltpu.make_async_copy(
                w_hbm.at[pl.ds(idx_ref[i], 1), :],
                sel_ref.at[pl.ds(i, 1), :],
                sem,
            ).start()
        for i in range(n_fields):
            pltpu.make_async_copy(
                w_hbm.at[pl.ds(idx_ref[i], 1), :],
                sel_ref.at[pl.ds(i, 1), :],
                sem,
            ).wait()

    sel = sel_ref[...]
    mean = jnp.mean(sel, axis=-1, keepdims=True)
    cent = sel - mean
    var = jnp.mean(cent * cent, axis=-1)
    c = cent * g_ref[...]
    v = v_ref[...]
    s = v * lax.rsqrt(v * v * var[None, :] + _EPS)
    out_ref[...] = s[:, :, None] * c[None] + b_ref[...][None]


def kernel(values, indices, W, ln_gamma, ln_beta):
    batch, n_fields = values.shape
    dim = W.shape[1]
    grid = batch // _B_BLK
    out = pl.pallas_call(
        _body,
        grid_spec=pltpu.PrefetchScalarGridSpec(
            num_scalar_prefetch=1,
            grid=(grid,),
            in_specs=[
                pl.BlockSpec((_B_BLK, n_fields), lambda i, idx: (i, 0)),
                pl.BlockSpec(memory_space=pl.ANY),
                pl.BlockSpec((1, dim), lambda i, idx: (0, 0)),
                pl.BlockSpec((1, dim), lambda i, idx: (0, 0)),
            ],
            out_specs=pl.BlockSpec(
                (_B_BLK, n_fields, dim), lambda i, idx: (i, 0, 0)),
            scratch_shapes=[
                pltpu.VMEM((n_fields, dim), jnp.float32),
                pltpu.SemaphoreType.DMA,
            ],
        ),
        out_shape=jax.ShapeDtypeStruct((batch, n_fields, dim), jnp.float32),
        compiler_params=pltpu.CompilerParams(
            dimension_semantics=("arbitrary",)),
    )(indices.astype(jnp.int32), values, W,
      ln_gamma.reshape(1, dim), ln_beta.reshape(1, dim))
    return out


# speedup vs baseline: 10.1439x; 6.3872x over previous
import jax
import jax.numpy as jnp
from jax import lax
from jax.experimental import pallas as pl
from jax.experimental.pallas import tpu as pltpu

_EPS = 1e-5
_LANES = 128


def _body(dv_ref, rm_ref, w_ref, v_ref, g_ref, b_ref, out_ref):
    n = pl.program_id(0)
    dim = w_ref.shape[0]
    blk = w_ref[...]
    lane = lax.broadcasted_iota(jnp.int32, (dim, _LANES), 1)
    col = jnp.sum(jnp.where(lane == rm_ref[n], blk, 0.0),
                  axis=1, keepdims=True)
    mean = jnp.sum(col, axis=0, keepdims=True) / dim
    cent = col - mean
    var = jnp.sum(cent * cent, axis=0, keepdims=True) / dim
    c = cent * g_ref[...]
    v = v_ref[...]
    s = v * lax.rsqrt(v * v * var[None] + _EPS)
    out_ref[...] = s * c[None] + b_ref[...][None]


def kernel(values, indices, W, ln_gamma, ln_beta):
    batch, n_fields = values.shape
    dim = W.shape[1]
    idx = indices.astype(jnp.int32)
    w_t = W.T
    v3 = values.T.reshape(n_fields, 1, batch)

    out3 = pl.pallas_call(
        _body,
        grid_spec=pltpu.PrefetchScalarGridSpec(
            num_scalar_prefetch=2,
            grid=(n_fields,),
            in_specs=[
                pl.BlockSpec((dim, _LANES), lambda n, dv, rm: (0, dv[n])),
                pl.BlockSpec((1, 1, batch), lambda n, dv, rm: (n, 0, 0)),
                pl.BlockSpec((dim, 1), lambda n, dv, rm: (0, 0)),
                pl.BlockSpec((dim, 1), lambda n, dv, rm: (0, 0)),
            ],
            out_specs=pl.BlockSpec(
                (1, dim, batch), lambda n, dv, rm: (n, 0, 0)),
        ),
        out_shape=jax.ShapeDtypeStruct((n_fields, dim, batch), jnp.float32),
        compiler_params=pltpu.CompilerParams(
            dimension_semantics=("arbitrary",)),
    )(idx // _LANES, idx % _LANES, w_t, v3,
      ln_gamma.reshape(dim, 1), ln_beta.reshape(dim, 1))
    return out3.transpose(2, 0, 1)


# speedup vs baseline: 20.3255x; 2.0037x over previous
import jax
import jax.numpy as jnp
from jax import lax
from jax.experimental import pallas as pl
from jax.experimental.pallas import tpu as pltpu

_EPS = 1e-5
_LANES = 128
_F = 10


def _body(dv_ref, rm_ref, *refs):
    n = pl.program_id(0)
    w_refs = refs[:_F]
    v_ref, g_ref, b_ref, out_ref = refs[_F:]
    dim = w_refs[0].shape[0]
    lane = lax.broadcasted_iota(jnp.int32, (dim, _LANES), 1)
    g = g_ref[...]
    b = b_ref[...]
    for k in range(_F):
        blk = w_refs[k][...]
        col = jnp.sum(jnp.where(lane == rm_ref[n * _F + k], blk, 0.0),
                      axis=1, keepdims=True)
        mean = jnp.sum(col, axis=0, keepdims=True) / dim
        cent = col - mean
        var = jnp.sum(cent * cent, axis=0, keepdims=True) / dim
        c = cent * g
        v = v_ref[k]
        s = v * lax.rsqrt(v * v * var + _EPS)
        out_ref[k] = s * c + b


def kernel(values, indices, W, ln_gamma, ln_beta):
    batch, n_fields = values.shape
    dim = W.shape[1]
    idx = indices.astype(jnp.int32)
    w_t = W.T
    v3 = values.T.reshape(n_fields, 1, batch)

    w_specs = [
        pl.BlockSpec((dim, _LANES),
                     lambda n, dv, rm, k=k: (0, dv[n * _F + k]))
        for k in range(_F)
    ]
    out3 = pl.pallas_call(
        _body,
        grid_spec=pltpu.PrefetchScalarGridSpec(
            num_scalar_prefetch=2,
            grid=(n_fields // _F,),
            in_specs=w_specs + [
                pl.BlockSpec((_F, 1, batch), lambda n, dv, rm: (n, 0, 0)),
                pl.BlockSpec((dim, 1), lambda n, dv, rm: (0, 0)),
                pl.BlockSpec((dim, 1), lambda n, dv, rm: (0, 0)),
            ],
            out_specs=pl.BlockSpec(
                (_F, dim, batch), lambda n, dv, rm: (n, 0, 0)),
        ),
        out_shape=jax.ShapeDtypeStruct((n_fields, dim, batch), jnp.float32),
        compiler_params=pltpu.CompilerParams(
            dimension_semantics=("arbitrary",)),
    )(idx // _LANES, idx % _LANES, *([w_t] * _F), v3,
      ln_gamma.reshape(dim, 1), ln_beta.reshape(dim, 1))
    return out3.transpose(2, 0, 1)
